# Initial kernel scaffold; baseline (speedup 1.0000x reference)
import jax
import jax.numpy as jnp
from jax.experimental import pallas as pl


def kernel(action_ids, embedding):
    raise NotImplementedError("write your pallas kernel here")


# speedup vs baseline: 2.7616x; 2.7616x over previous
import functools

import jax
import jax.numpy as jnp
from jax import lax
from jax.experimental import pallas as pl
from jax.experimental.pallas import tpu as pltpu
from jax.experimental.pallas import tpu_sc as plsc

D = 128
_info = plsc.get_sparse_core_info()
NC, NS = _info.num_cores, _info.num_subcores
NW = NC * NS
CHUNK = 128


@functools.partial(jax.jit, static_argnums=())
def _sc_gather(table, idx_flat):
    b_total = idx_flat.shape[0]
    b_per_w = b_total // NW
    n_chunks = b_per_w // CHUNK

    @functools.partial(
        pl.kernel,
        mesh=plsc.VectorSubcoreMesh(core_axis_name="c", subcore_axis_name="s"),
        out_type=jax.ShapeDtypeStruct((b_total, D), jnp.float32),
        scratch_types=[
            pltpu.VMEM((CHUNK,), jnp.int32),
            pltpu.VMEM((CHUNK, D), jnp.float32),
            pltpu.SemaphoreType.DMA,
        ],
    )
    def k(table_hbm, idx_hbm, out_hbm, idx_v, rows_v, sem):
        wid = lax.axis_index("s") * NC + lax.axis_index("c")
        base = wid * b_per_w

        def body(i, carry):
            off = base + i * CHUNK
            pltpu.sync_copy(idx_hbm.at[pl.ds(off, CHUNK)], idx_v)
            pltpu.async_copy(table_hbm.at[idx_v], rows_v, sem).wait()
            pltpu.sync_copy(rows_v, out_hbm.at[pl.ds(off, CHUNK)])
            return carry

        lax.fori_loop(0, n_chunks, body, 0)

    return k(table, idx_flat)


def kernel(action_ids, embedding):
    idx = action_ids.reshape(-1).astype(jnp.int32)
    out = _sc_gather(embedding, idx)
    return out.reshape(action_ids.shape + (D,))


# speedup vs baseline: 3.1338x; 1.1348x over previous
import functools

import jax
import jax.numpy as jnp
from jax import lax
from jax.experimental import pallas as pl
from jax.experimental.pallas import tpu as pltpu
from jax.experimental.pallas import tpu_sc as plsc

D = 128
_info = plsc.get_sparse_core_info()
NC, NS = _info.num_cores, _info.num_subcores
NW = NC * NS
CHUNK = 128


@jax.jit
def _sc_gather(table, idx_grouped):
    n_chunks = idx_grouped.shape[1]
    b_per_w = n_chunks * CHUNK
    b_total = NW * b_per_w
    n_pairs = n_chunks // 2

    @functools.partial(
        pl.kernel,
        mesh=plsc.VectorSubcoreMesh(core_axis_name="c", subcore_axis_name="s"),
        out_type=jax.ShapeDtypeStruct((b_total, D), jnp.float32),
        scratch_types=[
            pltpu.VMEM((n_chunks, CHUNK), jnp.int32),
            pltpu.VMEM((CHUNK, D), jnp.float32),
            pltpu.VMEM((CHUNK, D), jnp.float32),
            pltpu.SemaphoreType.DMA,
            pltpu.SemaphoreType.DMA,
            pltpu.SemaphoreType.DMA,
            pltpu.SemaphoreType.DMA,
        ],
    )
    def k(table_hbm, idx_hbm, out_hbm, idx_v, rows0, rows1, g0, g1, s0, s1):
        wid = lax.axis_index("s") * NC + lax.axis_index("c")
        base = wid * b_per_w
        pltpu.sync_copy(idx_hbm.at[wid], idx_v)

        def start_g(j, buf, sem):
            pltpu.async_copy(table_hbm.at[idx_v.at[j]], buf, sem)

        def wait_g(j, buf, sem):
            pltpu.make_async_copy(table_hbm.at[idx_v.at[j]], buf, sem).wait()

        def start_s(j, buf, sem):
            pltpu.async_copy(buf, out_hbm.at[pl.ds(base + j * CHUNK, CHUNK)], sem)

        def wait_s(j, buf, sem):
            pltpu.make_async_copy(
                buf, out_hbm.at[pl.ds(base + j * CHUNK, CHUNK)], sem
            ).wait()

        def pair(g, last):
            j0 = 2 * g
            j1 = j0 + 1
            wait_g(j0, rows0, g0)
            start_g(j1, rows1, g1)
            start_s(j0, rows0, s0)
            wait_g(j1, rows1, g1)
            start_s(j1, rows1, s1)
            wait_s(j0, rows0, s0)
            if not last:
                start_g(j0 + 2, rows0, g0)
            wait_s(j1, rows1, s1)

        start_g(0, rows0, g0)
        lax.fori_loop(0, n_pairs - 1, lambda g, c: (pair(g, False), c)[1], 0)
        pair(n_pairs - 1, True)

    return k(table, idx_grouped)


def kernel(action_ids, embedding):
    idx = action_ids.reshape(NW, -1, CHUNK).astype(jnp.int32)
    out = _sc_gather(embedding, idx)
    return out.reshape(action_ids.shape + (D,))


# speedup vs baseline: 3.3372x; 1.0649x over previous
import functools

import jax
import jax.numpy as jnp
from jax import lax
from jax.experimental import pallas as pl
from jax.experimental.pallas import tpu as pltpu
from jax.experimental.pallas import tpu_sc as plsc

D = 128
_info = plsc.get_sparse_core_info()
NC, NS = _info.num_cores, _info.num_subcores
NW = NC * NS
CHUNK = 128
NBUF = 4


@jax.jit
def _sc_gather(table, idx_grouped):
    n_chunks = idx_grouped.shape[1]
    b_per_w = n_chunks * CHUNK
    b_total = NW * b_per_w
    n_groups = n_chunks // NBUF

    @functools.partial(
        pl.kernel,
        mesh=plsc.VectorSubcoreMesh(core_axis_name="c", subcore_axis_name="s"),
        out_type=jax.ShapeDtypeStruct((b_total, D), jnp.float32),
        scratch_types=[
            pltpu.VMEM((n_chunks, CHUNK), jnp.int32),
            pltpu.VMEM((NBUF, CHUNK, D), jnp.float32),
            pltpu.SemaphoreType.DMA((NBUF,)),
            pltpu.SemaphoreType.DMA((NBUF,)),
        ],
    )
    def k(table_hbm, idx_hbm, out_hbm, idx_v, rows, gsem, ssem):
        wid = lax.axis_index("s") * NC + lax.axis_index("c")
        base = wid * b_per_w
        pltpu.sync_copy(idx_hbm.at[wid], idx_v)

        def start_g(j, b):
            pltpu.async_copy(table_hbm.at[idx_v.at[j]], rows.at[b], gsem.at[b])

        def wait_g(j, b):
            pltpu.make_async_copy(
                table_hbm.at[idx_v.at[j]], rows.at[b], gsem.at[b]
            ).wait()

        def start_s(j, b):
            pltpu.async_copy(
                rows.at[b], out_hbm.at[pl.ds(base + j * CHUNK, CHUNK)], ssem.at[b]
            )

        def wait_s(j, b):
            pltpu.make_async_copy(
                rows.at[b], out_hbm.at[pl.ds(base + j * CHUNK, CHUNK)], ssem.at[b]
            ).wait()

        def step(j, u, wait_prev, do_next):
            wait_g(j, u)
            start_s(j, u)
            if do_next:
                bp = (u - 1) % NBUF
                if wait_prev:
                    wait_s(j - 1, bp)
                start_g(j + NBUF - 1, bp)
            elif wait_prev:
                wait_s(j - 1, (u - 1) % NBUF)

        for u in range(NBUF - 1):
            start_g(u, u)
        for u in range(NBUF):
            step(u, u, wait_prev=(u > 0), do_next=True)
        def body(g, c):
            for u in range(NBUF):
                step(NBUF * g + u, u, wait_prev=True, do_next=True)
            return c
        lax.fori_loop(1, n_groups - 1, body, 0)
        jt = NBUF * (n_groups - 1)
        for u in range(NBUF):
            step(jt + u, u, wait_prev=True, do_next=(jt + u + NBUF - 1 < n_chunks))
        for j in range(NBUF * n_groups, n_chunks):
            step(j, j % NBUF, wait_prev=True, do_next=False)
        wait_s(n_chunks - 1, (n_chunks - 1) % NBUF)

    return k(table, idx_grouped)


def kernel(action_ids, embedding):
    idx = action_ids.reshape(NW, -1, CHUNK).astype(jnp.int32)
    out = _sc_gather(embedding, idx)
    return out.reshape(action_ids.shape + (D,))


# speedup vs baseline: 3.3465x; 1.0028x over previous
import functools

import jax
import jax.numpy as jnp
from jax import lax
from jax.experimental import pallas as pl
from jax.experimental.pallas import tpu as pltpu
from jax.experimental.pallas import tpu_sc as plsc

D = 128
_info = plsc.get_sparse_core_info()
NC, NS = _info.num_cores, _info.num_subcores
NW = NC * NS
CHUNK = 128
NBUF = 6


@jax.jit
def _sc_gather(table, idx_grouped):
    n_chunks = idx_grouped.shape[1]
    b_per_w = n_chunks * CHUNK
    b_total = NW * b_per_w
    n_groups = n_chunks // NBUF

    @functools.partial(
        pl.kernel,
        mesh=plsc.VectorSubcoreMesh(core_axis_name="c", subcore_axis_name="s"),
        out_type=jax.ShapeDtypeStruct((b_total, D), jnp.float32),
        scratch_types=[
            pltpu.VMEM((n_chunks, CHUNK), jnp.int32),
            pltpu.VMEM((NBUF, CHUNK, D), jnp.float32),
            pltpu.SemaphoreType.DMA((NBUF,)),
            pltpu.SemaphoreType.DMA((NBUF,)),
        ],
    )
    def k(table_hbm, idx_hbm, out_hbm, idx_v, rows, gsem, ssem):
        wid = lax.axis_index("s") * NC + lax.axis_index("c")
        base = wid * b_per_w
        pltpu.sync_copy(idx_hbm.at[wid], idx_v)

        def start_g(j, b):
            pltpu.async_copy(table_hbm.at[idx_v.at[j]], rows.at[b], gsem.at[b])

        def wait_g(j, b):
            pltpu.make_async_copy(
                table_hbm.at[idx_v.at[j]], rows.at[b], gsem.at[b]
            ).wait()

        def start_s(j, b):
            pltpu.async_copy(
                rows.at[b], out_hbm.at[pl.ds(base + j * CHUNK, CHUNK)], ssem.at[b]
            )

        def wait_s(j, b):
            pltpu.make_async_copy(
                rows.at[b], out_hbm.at[pl.ds(base + j * CHUNK, CHUNK)], ssem.at[b]
            ).wait()

        def step(j, u, wait_prev, do_next):
            wait_g(j, u)
            start_s(j, u)
            if do_next:
                bp = (u - 1) % NBUF
                if wait_prev:
                    wait_s(j - 1, bp)
                start_g(j + NBUF - 1, bp)
            elif wait_prev:
                wait_s(j - 1, (u - 1) % NBUF)

        for u in range(NBUF - 1):
            start_g(u, u)
        for u in range(NBUF):
            step(u, u, wait_prev=(u > 0), do_next=True)
        def body(g, c):
            for u in range(NBUF):
                step(NBUF * g + u, u, wait_prev=True, do_next=True)
            return c
        lax.fori_loop(1, n_groups - 1, body, 0)
        jt = NBUF * (n_groups - 1)
        for u in range(NBUF):
            step(jt + u, u, wait_prev=True, do_next=(jt + u + NBUF - 1 < n_chunks))
        for j in range(NBUF * n_groups, n_chunks):
            step(j, j % NBUF, wait_prev=True, do_next=False)
        wait_s(n_chunks - 1, (n_chunks - 1) % NBUF)

    return k(table, idx_grouped)


def kernel(action_ids, embedding):
    idx = action_ids.reshape(NW, -1, CHUNK).astype(jnp.int32)
    out = _sc_gather(embedding, idx)
    return out.reshape(action_ids.shape + (D,))


# speedup vs baseline: 5.9041x; 1.7642x over previous
import functools

import jax
import jax.numpy as jnp
from jax import lax
from jax.experimental import pallas as pl
from jax.experimental.pallas import tpu as pltpu
from jax.experimental.pallas import tpu_sc as plsc

D = 128
_info = plsc.get_sparse_core_info()
NC, NS = _info.num_cores, _info.num_subcores
NW = NC * NS
NBUF = 4


@jax.jit
def _sc_gather(table, idx):
    n_batch, n_pos = idx.shape
    b_per_w = n_batch // NW
    n_groups = b_per_w // NBUF

    @functools.partial(
        pl.kernel,
        mesh=plsc.VectorSubcoreMesh(core_axis_name="c", subcore_axis_name="s"),
        out_type=jax.ShapeDtypeStruct((n_batch, n_pos, D), jnp.float32),
        scratch_types=[
            pltpu.VMEM((b_per_w, n_pos), jnp.int32),
            pltpu.VMEM((NBUF, n_pos, D), jnp.float32),
            pltpu.SemaphoreType.DMA((NBUF,)),
            pltpu.SemaphoreType.DMA((NBUF,)),
        ],
    )
    def k(table_hbm, idx_hbm, out_hbm, idx_v, rows, gsem, ssem):
        wid = lax.axis_index("s") * NC + lax.axis_index("c")
        base = wid * b_per_w
        pltpu.sync_copy(idx_hbm.at[pl.ds(base, b_per_w)], idx_v)

        def start_g(j, b):
            pltpu.async_copy(table_hbm.at[idx_v.at[j]], rows.at[b], gsem.at[b])

        def wait_g(j, b):
            pltpu.make_async_copy(
                table_hbm.at[idx_v.at[j]], rows.at[b], gsem.at[b]
            ).wait()

        def start_s(j, b):
            pltpu.async_copy(rows.at[b], out_hbm.at[base + j], ssem.at[b])

        def wait_s(j, b):
            pltpu.make_async_copy(rows.at[b], out_hbm.at[base + j], ssem.at[b]).wait()

        def step(j, u, wait_prev, do_next):
            wait_g(j, u)
            start_s(j, u)
            if do_next:
                bp = (u - 1) % NBUF
                if wait_prev:
                    wait_s(j - 1, bp)
                start_g(j + NBUF - 1, bp)
            elif wait_prev:
                wait_s(j - 1, (u - 1) % NBUF)

        for u in range(NBUF - 1):
            start_g(u, u)
        for u in range(NBUF):
            step(u, u, wait_prev=(u > 0), do_next=True)
        def body(g, c):
            for u in range(NBUF):
                step(NBUF * g + u, u, wait_prev=True, do_next=True)
            return c
        lax.fori_loop(1, n_groups - 1, body, 0)
        jt = NBUF * (n_groups - 1)
        for u in range(NBUF):
            step(jt + u, u, wait_prev=True, do_next=(jt + u + NBUF - 1 < b_per_w))
        wait_s(b_per_w - 1, (b_per_w - 1) % NBUF)

    return k(table, idx)


def kernel(action_ids, embedding):
    return _sc_gather(embedding, action_ids.astype(jnp.int32))


# speedup vs baseline: 10.7204x; 1.8158x over previous
import functools

import jax
import jax.numpy as jnp
from jax import lax
from jax.experimental import pallas as pl
from jax.experimental.pallas import tpu as pltpu
from jax.experimental.pallas import tpu_sc as plsc

D = 128
_info = plsc.get_sparse_core_info()
NC, NS = _info.num_cores, _info.num_subcores
NW = NC * NS
NBUF = 4


@jax.jit
def _sc_gather(table, idx_t):
    n_pos, n_batch = idx_t.shape
    bpw = n_batch // NW
    n_chunks = n_pos
    n_groups = n_chunks // NBUF

    @functools.partial(
        pl.kernel,
        mesh=plsc.VectorSubcoreMesh(core_axis_name="c", subcore_axis_name="s"),
        out_type=jax.ShapeDtypeStruct((n_pos, n_batch, D), jnp.float32),
        scratch_types=[
            pltpu.VMEM((n_pos, bpw), jnp.int32),
            pltpu.VMEM((NBUF, bpw, D), jnp.float32),
            pltpu.SemaphoreType.DMA((NBUF,)),
            pltpu.SemaphoreType.DMA((NBUF,)),
        ],
    )
    def k(table_hbm, idx_hbm, out_hbm, idx_v, rows, gsem, ssem):
        wid = lax.axis_index("s") * NC + lax.axis_index("c")
        b0 = wid * bpw
        pltpu.sync_copy(idx_hbm.at[:, pl.ds(b0, bpw)], idx_v)

        def start_g(j, b):
            pltpu.async_copy(table_hbm.at[idx_v.at[j]], rows.at[b], gsem.at[b])

        def wait_g(j, b):
            pltpu.make_async_copy(
                table_hbm.at[idx_v.at[j]], rows.at[b], gsem.at[b]
            ).wait()

        def start_s(j, b):
            pltpu.async_copy(rows.at[b], out_hbm.at[j, pl.ds(b0, bpw)], ssem.at[b])

        def wait_s(j, b):
            pltpu.make_async_copy(
                rows.at[b], out_hbm.at[j, pl.ds(b0, bpw)], ssem.at[b]
            ).wait()

        def step(j, u, wait_prev, do_next):
            wait_g(j, u)
            start_s(j, u)
            if do_next:
                bp = (u - 1) % NBUF
                if wait_prev:
                    wait_s(j - 1, bp)
                start_g(j + NBUF - 1, bp)
            elif wait_prev:
                wait_s(j - 1, (u - 1) % NBUF)

        for u in range(NBUF - 1):
            start_g(u, u)
        for u in range(NBUF):
            step(u, u, wait_prev=(u > 0), do_next=True)
        def body(g, c):
            for u in range(NBUF):
                step(NBUF * g + u, u, wait_prev=True, do_next=True)
            return c
        lax.fori_loop(1, n_groups - 1, body, 0)
        jt = NBUF * (n_groups - 1)
        for u in range(NBUF):
            step(jt + u, u, wait_prev=True, do_next=(jt + u + NBUF - 1 < n_chunks))
        for j in range(NBUF * n_groups, n_chunks):
            step(j, j % NBUF, wait_prev=True, do_next=False)
        wait_s(n_chunks - 1, (n_chunks - 1) % NBUF)

    return k(table, idx_t)


def kernel(action_ids, embedding):
    idx_t = jnp.swapaxes(action_ids.astype(jnp.int32), 0, 1)
    out = _sc_gather(embedding, idx_t)
    return jnp.transpose(out, (1, 0, 2))
